# Initial kernel scaffold; baseline (speedup 1.0000x reference)
import jax
import jax.numpy as jnp
from jax.experimental import pallas as pl


def kernel(feature_map, keypoints):
    raise NotImplementedError("write your pallas kernel here")


# speedup vs baseline: 1.4523x; 1.4523x over previous
import functools

import numpy as np
import jax
import jax.numpy as jnp
from jax import lax
from jax.experimental import pallas as pl
from jax.experimental.pallas import tpu as pltpu
from jax.experimental.pallas import tpu_sc as plsc

_KS = 5
_SIGMA = 2.0
_HALF = _KS // 2

_NC = 2
_NS = 16
_NW = _NC * _NS
_L = 16
_IDX_CHUNK = 128


def _gauss1d():
    d = np.arange(-_HALF, _HALF + 1, dtype=np.float64)
    g = np.exp(-(d * d) / (2.0 * _SIGMA * _SIGMA))
    g = g / g.sum()
    return [float(v) for v in g]


_G = _gauss1d()


def _roll(v, shift, axis):
    if shift == 0:
        return v
    return jnp.roll(v, shift, axis)


def _blur_body(in_ref, out_ref):
    v = in_ref[...]
    acc = _G[0] * _roll(v, _HALF, 1)
    for k in range(1, _KS):
        acc += _G[k] * _roll(v, _HALF - k, 1)
    acc2 = _G[0] * _roll(acc, _HALF, 2)
    for k in range(1, _KS):
        acc2 += _G[k] * _roll(acc, _HALF - k, 2)
    out_ref[...] = acc2


def _tr_body(in_ref, out_ref):
    out_ref[...] = in_ref[...].T


def _make_gather(hw, c, n_pad):
    bpw = n_pad // _NW
    n_chunks = bpw // _IDX_CHUNK
    mesh = plsc.VectorSubcoreMesh(
        core_axis_name="c", subcore_axis_name="s",
        num_cores=_NC, num_subcores=_NS)

    @functools.partial(
        pl.kernel,
        mesh=mesh,
        compiler_params=pltpu.CompilerParams(use_tc_tiling_on_sc=False),
        out_type=jax.ShapeDtypeStruct((n_pad, c), jnp.float32),
        scratch_types=[
            pltpu.VMEM((bpw,), jnp.int32),
            pltpu.VMEM((bpw,), jnp.int32),
            pltpu.VMEM((n_chunks, _IDX_CHUNK), jnp.int32),
            pltpu.VMEM((bpw, c), jnp.float32),
            pltpu.SemaphoreType.DMA,
        ],
    )
    def gather_k(table_hbm, x_hbm, y_hbm, out_hbm, xv, yv, idxv, rows, sem):
        wid = lax.axis_index("s") * _NC + lax.axis_index("c")
        base = wid * bpw
        pltpu.sync_copy(x_hbm.at[pl.ds(base, bpw)], xv)
        pltpu.sync_copy(y_hbm.at[pl.ds(base, bpw)], yv)
        lo = jnp.int32(_HALF)
        hi = jnp.int32(511 - _HALF)
        for j in range(n_chunks):
            for kk in range(_IDX_CHUNK // _L):
                lane0 = j * _IDX_CHUNK + kk * _L
                xi = jnp.clip(xv[pl.ds(lane0, _L)], lo, hi)
                yi = jnp.clip(yv[pl.ds(lane0, _L)], lo, hi)
                idxv[j, pl.ds(kk * _L, _L)] = yi * jnp.int32(512) + xi
        copies = []
        for j in range(n_chunks):
            copies.append(pltpu.async_copy(
                table_hbm.at[idxv.at[j]],
                rows.at[pl.ds(j * _IDX_CHUNK, _IDX_CHUNK)],
                sem))
        for cp in copies:
            cp.wait()
        pltpu.sync_copy(rows, out_hbm.at[pl.ds(base, bpw)])

    return gather_k


def kernel(feature_map, keypoints):
    c, h, w = feature_map.shape
    n = keypoints.shape[0]

    cb = 2
    blurred = pl.pallas_call(
        _blur_body,
        grid=(c // cb,),
        in_specs=[pl.BlockSpec((cb, h, w), lambda i: (i, 0, 0))],
        out_specs=pl.BlockSpec((cb, h, w), lambda i: (i, 0, 0)),
        out_shape=jax.ShapeDtypeStruct((c, h, w), jnp.float32),
    )(feature_map)

    hw = h * w
    tb = 512
    table = pl.pallas_call(
        _tr_body,
        grid=(hw // tb,),
        in_specs=[pl.BlockSpec((c, tb), lambda i: (0, i))],
        out_specs=pl.BlockSpec((tb, c), lambda i: (i, 0)),
        out_shape=jax.ShapeDtypeStruct((hw, c), jnp.float32),
    )(blurred.reshape(c, hw))

    bpw = -(-n // _NW)
    bpw = -(-bpw // _IDX_CHUNK) * _IDX_CHUNK
    n_pad = bpw * _NW
    kp = keypoints.astype(jnp.int32)
    kp = jnp.pad(kp, ((0, n_pad - n), (0, 0)))
    xs = kp[:, 0]
    ys = kp[:, 1]

    out = _make_gather(hw, c, n_pad)(table, xs, ys)
    return out[:n]


# speedup vs baseline: 1.8683x; 1.2865x over previous
import functools

import numpy as np
import jax
import jax.numpy as jnp
from jax import lax
from jax.experimental import pallas as pl
from jax.experimental.pallas import tpu as pltpu
from jax.experimental.pallas import tpu_sc as plsc

_KS = 5
_SIGMA = 2.0
_HALF = _KS // 2

_NC = 2
_NS = 16
_NW = _NC * _NS
_L = 16
_IDX_CHUNK = 128


def _gauss1d():
    d = np.arange(-_HALF, _HALF + 1, dtype=np.float64)
    g = np.exp(-(d * d) / (2.0 * _SIGMA * _SIGMA))
    g = g / g.sum()
    return [float(v) for v in g]


_G = _gauss1d()


def _roll(v, shift, axis):
    if shift == 0:
        return v
    return jnp.roll(v, shift, axis)


def _blur_body(in_ref, out_ref):
    v = in_ref[...]
    acc = _G[0] * _roll(v, _HALF, 1)
    for k in range(1, _KS):
        acc += _G[k] * _roll(v, _HALF - k, 1)
    acc2 = _G[0] * _roll(acc, _HALF, 2)
    for k in range(1, _KS):
        acc2 += _G[k] * _roll(acc, _HALF - k, 2)
    out_ref[...] = acc2


def _tr_body(in_ref, out_ref):
    c = in_ref.shape[0]
    out_ref[:, pl.ds(0, c)] = in_ref[...].T


def _make_gather(hw, cp, n_pad):
    bpw = n_pad // _NW
    n_chunks = bpw // _IDX_CHUNK
    mesh = plsc.VectorSubcoreMesh(
        core_axis_name="c", subcore_axis_name="s",
        num_cores=_NC, num_subcores=_NS)

    @functools.partial(
        pl.kernel,
        mesh=mesh,
        out_type=jax.ShapeDtypeStruct((n_pad, cp), jnp.float32),
        scratch_types=[
            pltpu.VMEM((bpw,), jnp.int32),
            pltpu.VMEM((bpw,), jnp.int32),
            pltpu.VMEM((n_chunks, _IDX_CHUNK), jnp.int32),
            pltpu.VMEM((2, _IDX_CHUNK, cp), jnp.float32),
            pltpu.SemaphoreType.DMA,
            pltpu.SemaphoreType.DMA,
        ],
    )
    def gather_k(table_hbm, x_hbm, y_hbm, out_hbm, xv, yv, idxv, rows,
                 sem0, sem1):
        wid = lax.axis_index("s") * _NC + lax.axis_index("c")
        base = wid * bpw
        sems = (sem0, sem1)
        pltpu.sync_copy(x_hbm.at[pl.ds(base, bpw)], xv)
        pltpu.sync_copy(y_hbm.at[pl.ds(base, bpw)], yv)
        lo = jnp.int32(_HALF)
        hi = jnp.int32(511 - _HALF)
        copies = [None] * n_chunks
        for j in range(n_chunks):
            for kk in range(_IDX_CHUNK // _L):
                lane0 = j * _IDX_CHUNK + kk * _L
                xi = jnp.clip(xv[pl.ds(lane0, _L)], lo, hi)
                yi = jnp.clip(yv[pl.ds(lane0, _L)], lo, hi)
                idxv[j, pl.ds(kk * _L, _L)] = yi * jnp.int32(512) + xi
            copies[j] = pltpu.async_copy(
                table_hbm.at[idxv.at[j]], rows.at[j % 2], sems[j % 2])
            if j >= 1:
                copies[j - 1].wait()
                pltpu.sync_copy(
                    rows.at[(j - 1) % 2],
                    out_hbm.at[pl.ds(base + (j - 1) * _IDX_CHUNK, _IDX_CHUNK)])
        copies[n_chunks - 1].wait()
        pltpu.sync_copy(
            rows.at[(n_chunks - 1) % 2],
            out_hbm.at[pl.ds(base + (n_chunks - 1) * _IDX_CHUNK, _IDX_CHUNK)])

    return gather_k


def kernel(feature_map, keypoints):
    c, h, w = feature_map.shape
    n = keypoints.shape[0]

    cb = 2
    blurred = pl.pallas_call(
        _blur_body,
        grid=(c // cb,),
        in_specs=[pl.BlockSpec((cb, h, w), lambda i: (i, 0, 0))],
        out_specs=pl.BlockSpec((cb, h, w), lambda i: (i, 0, 0)),
        out_shape=jax.ShapeDtypeStruct((c, h, w), jnp.float32),
    )(feature_map)

    hw = h * w
    cp = 256
    tb = 512
    table = pl.pallas_call(
        _tr_body,
        grid=(hw // tb,),
        in_specs=[pl.BlockSpec((c, tb), lambda i: (0, i))],
        out_specs=pl.BlockSpec((tb, cp), lambda i: (i, 0)),
        out_shape=jax.ShapeDtypeStruct((hw, cp), jnp.float32),
    )(blurred.reshape(c, hw))

    bpw = -(-n // _NW)
    bpw = -(-bpw // _IDX_CHUNK) * _IDX_CHUNK
    n_pad = bpw * _NW
    kp = keypoints.astype(jnp.int32)
    kp = jnp.pad(kp, ((0, n_pad - n), (0, 0)))
    xs = kp[:, 0]
    ys = kp[:, 1]

    out = _make_gather(hw, cp, n_pad)(table, xs, ys)
    return out[:n, :c]


# speedup vs baseline: 2.9347x; 1.5707x over previous
import functools

import numpy as np
import jax
import jax.numpy as jnp
from jax import lax
from jax.experimental import pallas as pl
from jax.experimental.pallas import tpu as pltpu
from jax.experimental.pallas import tpu_sc as plsc

_KS = 5
_SIGMA = 2.0
_HALF = _KS // 2

_NC = 2
_NS = 16
_NW = _NC * _NS
_L = 16
_IDX_CHUNK = 128


def _gauss1d():
    d = np.arange(-_HALF, _HALF + 1, dtype=np.float64)
    g = np.exp(-(d * d) / (2.0 * _SIGMA * _SIGMA))
    g = g / g.sum()
    return [float(v) for v in g]


_G = _gauss1d()


def _roll(v, shift, axis):
    if shift == 0:
        return v
    return jnp.roll(v, shift, axis)


def _blur_body(in_ref, out_ref):
    v = in_ref[...]
    acc = _G[0] * _roll(v, _HALF, 1)
    for k in range(1, _KS):
        acc += _G[k] * _roll(v, _HALF - k, 1)
    acc2 = _G[0] * _roll(acc, _HALF, 2)
    for k in range(1, _KS):
        acc2 += _G[k] * _roll(acc, _HALF - k, 2)
    out_ref[...] = acc2


def _tr_body(in_ref, out_ref):
    c, hb, w = in_ref.shape
    for hl in range(hb):
        out_ref[pl.ds(hl * w, w), pl.ds(0, c)] = in_ref[:, hl, :].T


def _make_gather(hw, cp, n_pad):
    bpw = n_pad // _NW
    n_chunks = bpw // _IDX_CHUNK
    mesh = plsc.VectorSubcoreMesh(
        core_axis_name="c", subcore_axis_name="s",
        num_cores=_NC, num_subcores=_NS)

    @functools.partial(
        pl.kernel,
        mesh=mesh,
        compiler_params=pltpu.CompilerParams(use_tc_tiling_on_sc=True),
        out_type=jax.ShapeDtypeStruct((n_pad, cp), jnp.float32),
        scratch_types=[
            pltpu.VMEM((bpw,), jnp.int32),
            pltpu.VMEM((bpw,), jnp.int32),
            pltpu.VMEM((n_chunks, _IDX_CHUNK), jnp.int32),
            pltpu.VMEM((2, _IDX_CHUNK, cp), jnp.float32),
            pltpu.SemaphoreType.DMA,
            pltpu.SemaphoreType.DMA,
        ],
    )
    def gather_k(table_hbm, x_hbm, y_hbm, out_hbm, xv, yv, idxv, rows,
                 sem0, sem1):
        wid = lax.axis_index("s") * _NC + lax.axis_index("c")
        base = wid * bpw
        sems = (sem0, sem1)
        pltpu.sync_copy(x_hbm.at[pl.ds(base, bpw)], xv)
        pltpu.sync_copy(y_hbm.at[pl.ds(base, bpw)], yv)
        lo = jnp.int32(_HALF)
        hi = jnp.int32(511 - _HALF)
        copies = [None] * n_chunks
        for j in range(n_chunks):
            for kk in range(_IDX_CHUNK // _L):
                lane0 = j * _IDX_CHUNK + kk * _L
                xi = jnp.clip(xv[pl.ds(lane0, _L)], lo, hi)
                yi = jnp.clip(yv[pl.ds(lane0, _L)], lo, hi)
                idxv[j, pl.ds(kk * _L, _L)] = yi * jnp.int32(512) + xi
            copies[j] = pltpu.async_copy(
                table_hbm.at[idxv.at[j]], rows.at[j % 2], sems[j % 2])
            if j >= 1:
                copies[j - 1].wait()
                pltpu.sync_copy(
                    rows.at[(j - 1) % 2],
                    out_hbm.at[pl.ds(base + (j - 1) * _IDX_CHUNK, _IDX_CHUNK)])
        copies[n_chunks - 1].wait()
        pltpu.sync_copy(
            rows.at[(n_chunks - 1) % 2],
            out_hbm.at[pl.ds(base + (n_chunks - 1) * _IDX_CHUNK, _IDX_CHUNK)])

    return gather_k


def kernel(feature_map, keypoints):
    c, h, w = feature_map.shape
    n = keypoints.shape[0]

    cb = 2
    blurred = pl.pallas_call(
        _blur_body,
        grid=(c // cb,),
        in_specs=[pl.BlockSpec((cb, h, w), lambda i: (i, 0, 0))],
        out_specs=pl.BlockSpec((cb, h, w), lambda i: (i, 0, 0)),
        out_shape=jax.ShapeDtypeStruct((c, h, w), jnp.float32),
    )(feature_map)

    hw = h * w
    cp = 256
    hb = 8
    table = pl.pallas_call(
        _tr_body,
        grid=(h // hb,),
        in_specs=[pl.BlockSpec((c, hb, w), lambda i: (0, i, 0))],
        out_specs=pl.BlockSpec((hb * w, cp), lambda i: (i, 0)),
        out_shape=jax.ShapeDtypeStruct((hw, cp), jnp.float32),
    )(blurred)

    bpw = -(-n // _NW)
    bpw = -(-bpw // _IDX_CHUNK) * _IDX_CHUNK
    n_pad = bpw * _NW
    kp = keypoints.astype(jnp.int32)
    kp = jnp.pad(kp, ((0, n_pad - n), (0, 0)))
    xs = kp[:, 0]
    ys = kp[:, 1]

    out = _make_gather(hw, cp, n_pad)(table, xs, ys)
    return out[:n, :c]
